# Initial kernel scaffold; baseline (speedup 1.0000x reference)
import jax
import jax.numpy as jnp
from jax.experimental import pallas as pl


def kernel(x, edge_index_0, edge_index_1, W_self, W_r0, W_r1, b, W_out, b_out):
    raise NotImplementedError("write your pallas kernel here")


# speedup vs baseline: 5.1558x; 5.1558x over previous
import functools

import jax
import jax.numpy as jnp
from jax import lax
from jax.experimental import pallas as pl
from jax.experimental.pallas import tpu as pltpu
from jax.experimental.pallas import tpu_sc as plsc

N = 10000
D = 128
H = 128
C = 64
E = 160000

NUM_CORES = 2
NUM_SUBCORES = 16
EDGES_PER_TILE = E // NUM_SUBCORES
K = 80
NBLK = EDGES_PER_TILE // K
RB = 80
N_RBLK = N // RB
RBLK_PER_TILE = 8


def _sc_aggregate(x, src0, dst0, src1, dst1):
    mesh = plsc.VectorSubcoreMesh(core_axis_name="c", subcore_axis_name="s")

    @functools.partial(
        pl.kernel,
        out_type=(
            jax.ShapeDtypeStruct((N, D), jnp.float32),
            jax.ShapeDtypeStruct((N,), jnp.float32),
            jax.ShapeDtypeStruct((N, D), jnp.float32),
            jax.ShapeDtypeStruct((N,), jnp.float32),
        ),
        mesh=mesh,
        scratch_types=[
            pltpu.VMEM((K,), jnp.int32),
            pltpu.VMEM((K,), jnp.int32),
            pltpu.VMEM((K, D), jnp.float32),
            pltpu.VMEM((K,), jnp.float32),
            pltpu.VMEM((RB, D), jnp.float32),
            pltpu.VMEM((RB,), jnp.float32),
            pltpu.VMEM((RB,), jnp.float32),
            pltpu.VMEM_SHARED((N, D), jnp.float32),
            pltpu.VMEM_SHARED((N,), jnp.float32),
            pltpu.SemaphoreType.DMA,
        ],
    )
    def sc_kernel(x_hbm, s0_hbm, d0_hbm, s1_hbm, d1_hbm,
                  acc0_hbm, deg0_hbm, acc1_hbm, deg1_hbm,
                  srcv, dstv, rows, ones, zrows, zdeg, dbounce, accs, degs, sem):
        cid = lax.axis_index("c")
        sid = lax.axis_index("s")

        one16 = jnp.ones((16,), jnp.float32)
        zero16 = jnp.zeros((16,), jnp.float32)

        def fill_ones(i, _):
            ones[pl.ds(i * 16, 16)] = one16
            return 0
        lax.fori_loop(0, K // 16, fill_ones, 0)

        def fill_zdeg(i, _):
            zdeg[pl.ds(i * 16, 16)] = zero16
            return 0
        lax.fori_loop(0, RB // 16, fill_zdeg, 0)

        def fill_zrows(i, _):
            r = i // (D // 16)
            c = (i % (D // 16)) * 16
            zrows[r, pl.ds(c, 16)] = zero16
            return 0
        lax.fori_loop(0, RB * (D // 16), fill_zrows, 0)

        def run_relation(s_hbm, d_hbm, acc_hbm, deg_hbm):
            for j in range(RBLK_PER_TILE):
                bidx = j * NUM_SUBCORES + sid

                @pl.when(bidx < N_RBLK)
                def _():
                    off = pl.multiple_of(bidx * RB, RB)
                    pltpu.sync_copy(zrows, accs.at[pl.ds(off, RB)])
                    pltpu.sync_copy(zdeg, degs.at[pl.ds(off, RB)])

            plsc.subcore_barrier()

            def body(i, _):
                eoff = pl.multiple_of(sid * EDGES_PER_TILE + i * K, K)
                pltpu.sync_copy(s_hbm.at[pl.ds(eoff, K)], srcv)
                pltpu.sync_copy(d_hbm.at[pl.ds(eoff, K)], dstv)
                pltpu.async_copy(x_hbm.at[srcv], rows, sem).wait()
                pltpu.sync_copy(rows, accs.at[dstv], add=True)
                pltpu.sync_copy(ones, degs.at[dstv], add=True)
                return 0
            lax.fori_loop(0, NBLK, body, 0)

            plsc.subcore_barrier()

            for j in range(RBLK_PER_TILE):
                bidx = j * NUM_SUBCORES + sid

                @pl.when(bidx < N_RBLK)
                def _():
                    off = pl.multiple_of(bidx * RB, RB)
                    pltpu.sync_copy(accs.at[pl.ds(off, RB)], rows)
                    pltpu.sync_copy(rows, acc_hbm.at[pl.ds(off, RB)])
                    pltpu.sync_copy(degs.at[pl.ds(off, RB)], dbounce)
                    pltpu.sync_copy(dbounce, deg_hbm.at[pl.ds(off, RB)])

        @pl.when(cid == 0)
        def _():
            run_relation(s0_hbm, d0_hbm, acc0_hbm, deg0_hbm)

        @pl.when(cid == 1)
        def _():
            run_relation(s1_hbm, d1_hbm, acc1_hbm, deg1_hbm)

    return sc_kernel(x, src0, dst0, src1, dst1)


BLK = 1000


def _tc_body(x_ref, a0_ref, d0_ref, a1_ref, d1_ref,
             ws_ref, w0_ref, w1_ref, b_ref, wo_ref, bo_ref, out_ref):
    xb = x_ref[...]
    r0 = 1.0 / jnp.maximum(d0_ref[...], 1.0)
    r1 = 1.0 / jnp.maximum(d1_ref[...], 1.0)
    h = jnp.dot(xb, ws_ref[...], preferred_element_type=jnp.float32)
    h = h + jnp.dot(a0_ref[...] * r0, w0_ref[...], preferred_element_type=jnp.float32)
    h = h + jnp.dot(a1_ref[...] * r1, w1_ref[...], preferred_element_type=jnp.float32)
    h = jnp.maximum(h + b_ref[...], 0.0)
    logits = jnp.dot(h, wo_ref[...], preferred_element_type=jnp.float32) + bo_ref[...]
    m = jnp.max(logits, axis=1, keepdims=True)
    e = jnp.exp(logits - m)
    out_ref[...] = e / jnp.sum(e, axis=1, keepdims=True)


def _tc_head(x, acc0, deg0, acc1, deg1, W_self, W_r0, W_r1, b, W_out, b_out):
    grid = (N // BLK,)
    row = lambda i: (i, 0)
    full = lambda i: (0, 0)
    return pl.pallas_call(
        _tc_body,
        grid=grid,
        in_specs=[
            pl.BlockSpec((BLK, D), row),
            pl.BlockSpec((BLK, D), row),
            pl.BlockSpec((BLK, 1), row),
            pl.BlockSpec((BLK, D), row),
            pl.BlockSpec((BLK, 1), row),
            pl.BlockSpec((D, H), full),
            pl.BlockSpec((D, H), full),
            pl.BlockSpec((D, H), full),
            pl.BlockSpec((1, H), full),
            pl.BlockSpec((H, C), full),
            pl.BlockSpec((1, C), full),
        ],
        out_specs=pl.BlockSpec((BLK, C), row),
        out_shape=jax.ShapeDtypeStruct((N, C), jnp.float32),
    )(x, acc0, deg0, acc1, deg1, W_self, W_r0, W_r1, b, W_out, b_out)


def kernel(x, edge_index_0, edge_index_1, W_self, W_r0, W_r1, b, W_out, b_out):
    acc0, deg0, acc1, deg1 = _sc_aggregate(
        x, edge_index_0[0], edge_index_0[1], edge_index_1[0], edge_index_1[1])
    return _tc_head(
        x, acc0, deg0.reshape(N, 1), acc1, deg1.reshape(N, 1),
        W_self, W_r0, W_r1, b.reshape(1, H), W_out, b_out.reshape(1, C))


# speedup vs baseline: 10.7776x; 2.0904x over previous
import functools

import jax
import jax.numpy as jnp
from jax import lax
from jax.experimental import pallas as pl
from jax.experimental.pallas import tpu as pltpu
from jax.experimental.pallas import tpu_sc as plsc

N = 10000
D = 128
H = 128
C = 64
E = 160000

NUM_CORES = 2
NUM_SUBCORES = 16
EDGES_PER_TILE = E // NUM_SUBCORES
K = 125
NBLK = EDGES_PER_TILE // K
RB = 80
N_RBLK = N // RB
RBLK_PER_TILE = 8


def _sc_aggregate(x, src0, dst0, src1, dst1):
    mesh = plsc.VectorSubcoreMesh(core_axis_name="c", subcore_axis_name="s")

    @functools.partial(
        pl.kernel,
        out_type=(
            jax.ShapeDtypeStruct((N, D), jnp.float32),
            jax.ShapeDtypeStruct((N,), jnp.float32),
            jax.ShapeDtypeStruct((N, D), jnp.float32),
            jax.ShapeDtypeStruct((N,), jnp.float32),
        ),
        mesh=mesh,
        scratch_types=[
            pltpu.VMEM((4, K), jnp.int32),
            pltpu.VMEM((4, K), jnp.int32),
            pltpu.VMEM((K, D), jnp.float32),
            pltpu.VMEM((K, D), jnp.float32),
            pltpu.VMEM((K,), jnp.float32),
            pltpu.VMEM((RB,), jnp.float32),
            pltpu.VMEM_SHARED((N, D), jnp.float32),
            pltpu.VMEM_SHARED((N,), jnp.float32),
            [pltpu.SemaphoreType.DMA] * 4,
            [pltpu.SemaphoreType.DMA] * 2,
        ],
    )
    def sc_kernel(x_hbm, s0_hbm, d0_hbm, s1_hbm, d1_hbm,
                  acc0_hbm, deg0_hbm, acc1_hbm, deg1_hbm,
                  sidx, didx, rows0, rows1, ones, zdeg,
                  accs, degs, isems, gsems):
        cid = lax.axis_index("c")
        sid = lax.axis_index("s")

        zero16 = jnp.zeros((16,), jnp.float32)
        one16 = jnp.ones((16,), jnp.float32)

        def fill_ones(i, _):
            ones[pl.ds(jnp.minimum(i * 16, K - 16), 16)] = one16
            return 0
        lax.fori_loop(0, (K + 15) // 16, fill_ones, 0)

        rowslots = (rows0, rows1)

        def run_relation(s_hbm, d_hbm, acc_hbm, deg_hbm):
            def fill_zdeg(i, _):
                zdeg[pl.ds(i * 16, 16)] = zero16
                return 0
            lax.fori_loop(0, RB // 16, fill_zdeg, 0)

            def fill_zrows(i, _):
                r = i // (D // 16)
                c = (i % (D // 16)) * 16
                rows0[r, pl.ds(c, 16)] = zero16
                return 0
            lax.fori_loop(0, RB * (D // 16), fill_zrows, 0)

            for j in range(RBLK_PER_TILE):
                bidx = j * NUM_SUBCORES + sid

                @pl.when(bidx < N_RBLK)
                def _():
                    off = pl.multiple_of(bidx * RB, RB)
                    pltpu.sync_copy(rows0.at[pl.ds(0, RB)], accs.at[pl.ds(off, RB)])
                    pltpu.sync_copy(zdeg, degs.at[pl.ds(off, RB)])

            plsc.subcore_barrier()

            def idx_load(blk, slot):
                pltpu.async_copy(s_hbm.at[sid, blk], sidx.at[slot], isems[slot])
                pltpu.async_copy(d_hbm.at[sid, blk], didx.at[slot], isems[slot])

            def wait_idx(blk, slot):
                pltpu.make_async_copy(s_hbm.at[sid, blk], sidx.at[slot], isems[slot]).wait()
                pltpu.make_async_copy(d_hbm.at[sid, blk], didx.at[slot], isems[slot]).wait()

            def gather(rows, slot, g):
                pltpu.async_copy(x_hbm.at[sidx.at[slot]], rows, gsems[g])

            def wait_gather(rows, slot, g):
                pltpu.make_async_copy(x_hbm.at[sidx.at[slot]], rows, gsems[g]).wait()

            for t in range(4):
                idx_load(t, t)
            wait_idx(0, 0)
            gather(rows0, 0, 0)
            wait_idx(1, 1)
            gather(rows1, 1, 1)

            def body(j, _):
                for t in range(4):
                    b = j * 4 + t
                    rows = rowslots[t % 2]
                    g = t % 2
                    wait_gather(rows, t, g)
                    pltpu.sync_copy(rows, accs.at[didx.at[t]], add=True)
                    pltpu.sync_copy(ones, degs.at[didx.at[t]], add=True)

                    t2 = (t + 2) % 4

                    @pl.when(b + 2 < NBLK)
                    def _():
                        wait_idx(b + 2, t2)
                        gather(rows, t2, g)

                    @pl.when(b + 4 < NBLK)
                    def _():
                        idx_load(b + 4, t)
                return 0
            lax.fori_loop(0, NBLK // 4, body, 0)

            plsc.subcore_barrier()

            for j in range(RBLK_PER_TILE):
                bidx = j * NUM_SUBCORES + sid

                @pl.when(bidx < N_RBLK)
                def _():
                    off = pl.multiple_of(bidx * RB, RB)
                    pltpu.sync_copy(accs.at[pl.ds(off, RB)], rows0.at[pl.ds(0, RB)])
                    pltpu.sync_copy(rows0.at[pl.ds(0, RB)], acc_hbm.at[pl.ds(off, RB)])
                    pltpu.sync_copy(degs.at[pl.ds(off, RB)], zdeg)
                    pltpu.sync_copy(zdeg, deg_hbm.at[pl.ds(off, RB)])

        @pl.when(cid == 0)
        def _():
            run_relation(s0_hbm, d0_hbm, acc0_hbm, deg0_hbm)

        @pl.when(cid == 1)
        def _():
            run_relation(s1_hbm, d1_hbm, acc1_hbm, deg1_hbm)

    return sc_kernel(x, src0, dst0, src1, dst1)


BLK = 1000


def _tc_body(x_ref, a0_ref, d0_ref, a1_ref, d1_ref,
             ws_ref, w0_ref, w1_ref, b_ref, wo_ref, bo_ref, out_ref):
    xb = x_ref[...]
    r0 = 1.0 / jnp.maximum(d0_ref[...], 1.0)
    r1 = 1.0 / jnp.maximum(d1_ref[...], 1.0)
    h = jnp.dot(xb, ws_ref[...], preferred_element_type=jnp.float32)
    h = h + jnp.dot(a0_ref[...] * r0, w0_ref[...], preferred_element_type=jnp.float32)
    h = h + jnp.dot(a1_ref[...] * r1, w1_ref[...], preferred_element_type=jnp.float32)
    h = jnp.maximum(h + b_ref[...], 0.0)
    logits = jnp.dot(h, wo_ref[...], preferred_element_type=jnp.float32) + bo_ref[...]
    m = jnp.max(logits, axis=1, keepdims=True)
    e = jnp.exp(logits - m)
    out_ref[...] = e / jnp.sum(e, axis=1, keepdims=True)


def _tc_head(x, acc0, deg0, acc1, deg1, W_self, W_r0, W_r1, b, W_out, b_out):
    grid = (N // BLK,)
    row = lambda i: (i, 0)
    full = lambda i: (0, 0)
    return pl.pallas_call(
        _tc_body,
        grid=grid,
        in_specs=[
            pl.BlockSpec((BLK, D), row),
            pl.BlockSpec((BLK, D), row),
            pl.BlockSpec((BLK, 1), row),
            pl.BlockSpec((BLK, D), row),
            pl.BlockSpec((BLK, 1), row),
            pl.BlockSpec((D, H), full),
            pl.BlockSpec((D, H), full),
            pl.BlockSpec((D, H), full),
            pl.BlockSpec((1, H), full),
            pl.BlockSpec((H, C), full),
            pl.BlockSpec((1, C), full),
        ],
        out_specs=pl.BlockSpec((BLK, C), row),
        out_shape=jax.ShapeDtypeStruct((N, C), jnp.float32),
    )(x, acc0, deg0, acc1, deg1, W_self, W_r0, W_r1, b, W_out, b_out)


def kernel(x, edge_index_0, edge_index_1, W_self, W_r0, W_r1, b, W_out, b_out):
    acc0, deg0, acc1, deg1 = _sc_aggregate(
        x,
        edge_index_0[0].reshape(NUM_SUBCORES, NBLK, K),
        edge_index_0[1].reshape(NUM_SUBCORES, NBLK, K),
        edge_index_1[0].reshape(NUM_SUBCORES, NBLK, K),
        edge_index_1[1].reshape(NUM_SUBCORES, NBLK, K),
    )
    return _tc_head(
        x, acc0, deg0.reshape(N, 1), acc1, deg1.reshape(N, 1),
        W_self, W_r0, W_r1, b.reshape(1, H), W_out, b_out.reshape(1, C))


# speedup vs baseline: 10.8104x; 1.0030x over previous
import functools

import jax
import jax.numpy as jnp
from jax import lax
from jax.experimental import pallas as pl
from jax.experimental.pallas import tpu as pltpu
from jax.experimental.pallas import tpu_sc as plsc

N = 10000
D = 128
H = 128
C = 64
E = 160000

NUM_CORES = 2
NUM_SUBCORES = 16
EDGES_PER_TILE = E // NUM_SUBCORES
K = 125
NBLK = EDGES_PER_TILE // K
RB = 80
N_RBLK = N // RB
RBLK_PER_TILE = 8


def _sc_aggregate(x, src0, dst0, src1, dst1):
    mesh = plsc.VectorSubcoreMesh(core_axis_name="c", subcore_axis_name="s")

    @functools.partial(
        pl.kernel,
        out_type=(
            jax.ShapeDtypeStruct((N, D), jnp.float32),
            jax.ShapeDtypeStruct((N,), jnp.float32),
            jax.ShapeDtypeStruct((N, D), jnp.float32),
            jax.ShapeDtypeStruct((N,), jnp.float32),
        ),
        mesh=mesh,
        scratch_types=[
            pltpu.VMEM((4, K), jnp.int32),
            pltpu.VMEM((4, K), jnp.int32),
            pltpu.VMEM((K, D), jnp.float32),
            pltpu.VMEM((K, D), jnp.float32),
            pltpu.VMEM((K,), jnp.float32),
            pltpu.VMEM((RB,), jnp.float32),
            pltpu.VMEM_SHARED((N, D), jnp.float32),
            pltpu.VMEM_SHARED((N,), jnp.float32),
            [pltpu.SemaphoreType.DMA] * 4,
            [pltpu.SemaphoreType.DMA] * 2,
            [pltpu.SemaphoreType.DMA] * 2,
            [pltpu.SemaphoreType.DMA] * 2,
        ],
    )
    def sc_kernel(x_hbm, s0_hbm, d0_hbm, s1_hbm, d1_hbm,
                  acc0_hbm, deg0_hbm, acc1_hbm, deg1_hbm,
                  sidx, didx, rows0, rows1, ones, zdeg,
                  accs, degs, isems, gsems, ssems, dsems):
        cid = lax.axis_index("c")
        sid = lax.axis_index("s")

        zero16 = jnp.zeros((16,), jnp.float32)
        one16 = jnp.ones((16,), jnp.float32)

        def fill_ones(i, _):
            ones[pl.ds(jnp.minimum(i * 16, K - 16), 16)] = one16
            return 0
        lax.fori_loop(0, (K + 15) // 16, fill_ones, 0)

        rowslots = (rows0, rows1)

        def run_relation(s_hbm, d_hbm, acc_hbm, deg_hbm):
            def fill_zdeg(i, _):
                zdeg[pl.ds(i * 16, 16)] = zero16
                return 0
            lax.fori_loop(0, RB // 16, fill_zdeg, 0)

            def fill_zrows(i, _):
                r = i // (D // 16)
                c = (i % (D // 16)) * 16
                rows0[r, pl.ds(c, 16)] = zero16
                return 0
            lax.fori_loop(0, RB * (D // 16), fill_zrows, 0)

            for j in range(RBLK_PER_TILE):
                bidx = j * NUM_SUBCORES + sid

                @pl.when(bidx < N_RBLK)
                def _():
                    off = pl.multiple_of(bidx * RB, RB)
                    pltpu.sync_copy(rows0.at[pl.ds(0, RB)], accs.at[pl.ds(off, RB)])
                    pltpu.sync_copy(zdeg, degs.at[pl.ds(off, RB)])

            plsc.subcore_barrier()

            def idx_load(blk, slot):
                pltpu.async_copy(s_hbm.at[sid, blk], sidx.at[slot], isems[slot])
                pltpu.async_copy(d_hbm.at[sid, blk], didx.at[slot], isems[slot])

            def wait_idx(blk, slot):
                pltpu.make_async_copy(s_hbm.at[sid, blk], sidx.at[slot], isems[slot]).wait()
                pltpu.make_async_copy(d_hbm.at[sid, blk], didx.at[slot], isems[slot]).wait()

            def gather(rows, slot, g):
                pltpu.async_copy(x_hbm.at[sidx.at[slot]], rows, gsems[g])

            def wait_gather(rows, slot, g):
                pltpu.make_async_copy(x_hbm.at[sidx.at[slot]], rows, gsems[g]).wait()

            for t in range(4):
                idx_load(t, t)
            wait_idx(0, 0)
            gather(rows0, 0, 0)
            wait_idx(1, 1)
            gather(rows1, 1, 1)

            def body(j, _):
                for t in range(4):
                    b = j * 4 + t
                    rows = rowslots[t % 2]
                    g = t % 2
                    wait_gather(rows, t, g)
                    pltpu.async_copy(rows, accs.at[didx.at[t]], ssems[g], add=True)
                    pltpu.async_copy(ones, degs.at[didx.at[t]], dsems[g], add=True)

                    t2 = (t + 2) % 4

                    @pl.when(b + 2 < NBLK)
                    def _():
                        wait_idx(b + 2, t2)

                    pltpu.make_async_copy(rows, accs.at[didx.at[t]], ssems[g]).wait()

                    @pl.when(b + 2 < NBLK)
                    def _():
                        gather(rows, t2, g)

                    pltpu.make_async_copy(ones, degs.at[didx.at[t]], dsems[g]).wait()

                    @pl.when(b + 4 < NBLK)
                    def _():
                        idx_load(b + 4, t)
                return 0
            lax.fori_loop(0, NBLK // 4, body, 0)

            plsc.subcore_barrier()

            for j in range(RBLK_PER_TILE):
                bidx = j * NUM_SUBCORES + sid

                @pl.when(bidx < N_RBLK)
                def _():
                    off = pl.multiple_of(bidx * RB, RB)
                    pltpu.sync_copy(accs.at[pl.ds(off, RB)], rows0.at[pl.ds(0, RB)])
                    pltpu.sync_copy(rows0.at[pl.ds(0, RB)], acc_hbm.at[pl.ds(off, RB)])
                    pltpu.sync_copy(degs.at[pl.ds(off, RB)], zdeg)
                    pltpu.sync_copy(zdeg, deg_hbm.at[pl.ds(off, RB)])

        @pl.when(cid == 0)
        def _():
            run_relation(s0_hbm, d0_hbm, acc0_hbm, deg0_hbm)

        @pl.when(cid == 1)
        def _():
            run_relation(s1_hbm, d1_hbm, acc1_hbm, deg1_hbm)

    return sc_kernel(x, src0, dst0, src1, dst1)


BLK = 1000


def _tc_body(x_ref, a0_ref, d0_ref, a1_ref, d1_ref,
             ws_ref, w0_ref, w1_ref, b_ref, wo_ref, bo_ref, out_ref):
    xb = x_ref[...]
    r0 = 1.0 / jnp.maximum(d0_ref[...], 1.0)
    r1 = 1.0 / jnp.maximum(d1_ref[...], 1.0)
    h = jnp.dot(xb, ws_ref[...], preferred_element_type=jnp.float32)
    h = h + jnp.dot(a0_ref[...] * r0, w0_ref[...], preferred_element_type=jnp.float32)
    h = h + jnp.dot(a1_ref[...] * r1, w1_ref[...], preferred_element_type=jnp.float32)
    h = jnp.maximum(h + b_ref[...], 0.0)
    logits = jnp.dot(h, wo_ref[...], preferred_element_type=jnp.float32) + bo_ref[...]
    m = jnp.max(logits, axis=1, keepdims=True)
    e = jnp.exp(logits - m)
    out_ref[...] = e / jnp.sum(e, axis=1, keepdims=True)


def _tc_head(x, acc0, deg0, acc1, deg1, W_self, W_r0, W_r1, b, W_out, b_out):
    grid = (N // BLK,)
    row = lambda i: (i, 0)
    full = lambda i: (0, 0)
    return pl.pallas_call(
        _tc_body,
        grid=grid,
        in_specs=[
            pl.BlockSpec((BLK, D), row),
            pl.BlockSpec((BLK, D), row),
            pl.BlockSpec((BLK, 1), row),
            pl.BlockSpec((BLK, D), row),
            pl.BlockSpec((BLK, 1), row),
            pl.BlockSpec((D, H), full),
            pl.BlockSpec((D, H), full),
            pl.BlockSpec((D, H), full),
            pl.BlockSpec((1, H), full),
            pl.BlockSpec((H, C), full),
            pl.BlockSpec((1, C), full),
        ],
        out_specs=pl.BlockSpec((BLK, C), row),
        out_shape=jax.ShapeDtypeStruct((N, C), jnp.float32),
    )(x, acc0, deg0, acc1, deg1, W_self, W_r0, W_r1, b, W_out, b_out)


def kernel(x, edge_index_0, edge_index_1, W_self, W_r0, W_r1, b, W_out, b_out):
    acc0, deg0, acc1, deg1 = _sc_aggregate(
        x,
        edge_index_0[0].reshape(NUM_SUBCORES, NBLK, K),
        edge_index_0[1].reshape(NUM_SUBCORES, NBLK, K),
        edge_index_1[0].reshape(NUM_SUBCORES, NBLK, K),
        edge_index_1[1].reshape(NUM_SUBCORES, NBLK, K),
    )
    return _tc_head(
        x, acc0, deg0.reshape(N, 1), acc1, deg1.reshape(N, 1),
        W_self, W_r0, W_r1, b.reshape(1, H), W_out, b_out.reshape(1, C))


# speedup vs baseline: 11.7752x; 1.0892x over previous
import functools

import jax
import jax.numpy as jnp
from jax import lax
from jax.experimental import pallas as pl
from jax.experimental.pallas import tpu as pltpu
from jax.experimental.pallas import tpu_sc as plsc

N = 10000
D = 128
H = 128
C = 64
E = 160000

NUM_CORES = 2
NUM_SUBCORES = 16
EDGES_PER_TILE = E // NUM_SUBCORES
K = 125
NBLK = EDGES_PER_TILE // K
RB = 80
N_RBLK = N // RB
RBLK_PER_TILE = 8


def _sc_aggregate(x, e0, e1):
    mesh = plsc.VectorSubcoreMesh(core_axis_name="c", subcore_axis_name="s")

    @functools.partial(
        pl.kernel,
        out_type=(
            jax.ShapeDtypeStruct((N, D), jnp.float32),
            jax.ShapeDtypeStruct((N,), jnp.float32),
            jax.ShapeDtypeStruct((N, D), jnp.float32),
            jax.ShapeDtypeStruct((N,), jnp.float32),
        ),
        mesh=mesh,
        scratch_types=[
            pltpu.VMEM((4, K), jnp.int32),
            pltpu.VMEM((4, K), jnp.int32),
            pltpu.VMEM((K, D), jnp.float32),
            pltpu.VMEM((K, D), jnp.float32),
            pltpu.VMEM((K,), jnp.float32),
            pltpu.VMEM((RB,), jnp.float32),
            pltpu.VMEM_SHARED((N, D), jnp.float32),
            pltpu.VMEM_SHARED((N,), jnp.float32),
            [pltpu.SemaphoreType.DMA] * 4,
            [pltpu.SemaphoreType.DMA] * 2,
            [pltpu.SemaphoreType.DMA] * 2,
            [pltpu.SemaphoreType.DMA] * 2,
        ],
    )
    def sc_kernel(x_hbm, e0_hbm, e1_hbm,
                  acc0_hbm, deg0_hbm, acc1_hbm, deg1_hbm,
                  sidx, didx, rows0, rows1, ones, zdeg,
                  accs, degs, isems, gsems, ssems, dsems):
        cid = lax.axis_index("c")
        sid = lax.axis_index("s")

        zero16 = jnp.zeros((16,), jnp.float32)
        one16 = jnp.ones((16,), jnp.float32)

        def fill_ones(i, _):
            ones[pl.ds(jnp.minimum(i * 16, K - 16), 16)] = one16
            return 0
        lax.fori_loop(0, (K + 15) // 16, fill_ones, 0)

        rowslots = (rows0, rows1)

        def run_relation(e_hbm, acc_hbm, deg_hbm):
            def fill_zdeg(i, _):
                zdeg[pl.ds(i * 16, 16)] = zero16
                return 0
            lax.fori_loop(0, RB // 16, fill_zdeg, 0)

            def fill_zrows(i, _):
                r = i // (D // 16)
                c = (i % (D // 16)) * 16
                rows0[r, pl.ds(c, 16)] = zero16
                return 0
            lax.fori_loop(0, RB * (D // 16), fill_zrows, 0)

            for j in range(RBLK_PER_TILE):
                bidx = j * NUM_SUBCORES + sid

                @pl.when(bidx < N_RBLK)
                def _():
                    off = pl.multiple_of(bidx * RB, RB)
                    pltpu.sync_copy(rows0.at[pl.ds(0, RB)], accs.at[pl.ds(off, RB)])
                    pltpu.sync_copy(zdeg, degs.at[pl.ds(off, RB)])

            plsc.subcore_barrier()

            def idx_load(blk, slot):
                pltpu.async_copy(e_hbm.at[0, sid, blk], sidx.at[slot], isems[slot])
                pltpu.async_copy(e_hbm.at[1, sid, blk], didx.at[slot], isems[slot])

            def wait_idx(blk, slot):
                pltpu.make_async_copy(e_hbm.at[0, sid, blk], sidx.at[slot], isems[slot]).wait()
                pltpu.make_async_copy(e_hbm.at[1, sid, blk], didx.at[slot], isems[slot]).wait()

            def gather(rows, slot, g):
                pltpu.async_copy(x_hbm.at[sidx.at[slot]], rows, gsems[g])

            def wait_gather(rows, slot, g):
                pltpu.make_async_copy(x_hbm.at[sidx.at[slot]], rows, gsems[g]).wait()

            for t in range(4):
                idx_load(t, t)
            wait_idx(0, 0)
            gather(rows0, 0, 0)
            wait_idx(1, 1)
            gather(rows1, 1, 1)

            def body(j, _):
                for t in range(4):
                    b = j * 4 + t
                    rows = rowslots[t % 2]
                    g = t % 2
                    wait_gather(rows, t, g)
                    pltpu.async_copy(rows, accs.at[didx.at[t]], ssems[g], add=True)
                    pltpu.async_copy(ones, degs.at[didx.at[t]], dsems[g], add=True)

                    t2 = (t + 2) % 4

                    @pl.when(b + 2 < NBLK)
                    def _():
                        wait_idx(b + 2, t2)

                    pltpu.make_async_copy(rows, accs.at[didx.at[t]], ssems[g]).wait()

                    @pl.when(b + 2 < NBLK)
                    def _():
                        gather(rows, t2, g)

                    pltpu.make_async_copy(ones, degs.at[didx.at[t]], dsems[g]).wait()

                    @pl.when(b + 4 < NBLK)
                    def _():
                        idx_load(b + 4, t)
                return 0
            lax.fori_loop(0, NBLK // 4, body, 0)

            plsc.subcore_barrier()

            for j in range(RBLK_PER_TILE):
                bidx = j * NUM_SUBCORES + sid

                @pl.when(bidx < N_RBLK)
                def _():
                    off = pl.multiple_of(bidx * RB, RB)
                    pltpu.sync_copy(accs.at[pl.ds(off, RB)], rows0.at[pl.ds(0, RB)])
                    pltpu.sync_copy(rows0.at[pl.ds(0, RB)], acc_hbm.at[pl.ds(off, RB)])
                    pltpu.sync_copy(degs.at[pl.ds(off, RB)], zdeg)
                    pltpu.sync_copy(zdeg, deg_hbm.at[pl.ds(off, RB)])

        @pl.when(cid == 0)
        def _():
            run_relation(e0_hbm, acc0_hbm, deg0_hbm)

        @pl.when(cid == 1)
        def _():
            run_relation(e1_hbm, acc1_hbm, deg1_hbm)

    return sc_kernel(x, e0, e1)


BLK = 2000


def _tc_body(x_ref, a0_ref, d0_ref, a1_ref, d1_ref,
             ws_ref, w0_ref, w1_ref, b_ref, wo_ref, bo_ref, out_ref):
    xb = x_ref[...]
    r0 = 1.0 / jnp.maximum(d0_ref[...], 1.0)
    r1 = 1.0 / jnp.maximum(d1_ref[...], 1.0)
    h = jnp.dot(xb, ws_ref[...], preferred_element_type=jnp.float32)
    h = h + jnp.dot(a0_ref[...] * r0, w0_ref[...], preferred_element_type=jnp.float32)
    h = h + jnp.dot(a1_ref[...] * r1, w1_ref[...], preferred_element_type=jnp.float32)
    h = jnp.maximum(h + b_ref[...], 0.0)
    logits = jnp.dot(h, wo_ref[...], preferred_element_type=jnp.float32) + bo_ref[...]
    m = jnp.max(logits, axis=1, keepdims=True)
    e = jnp.exp(logits - m)
    out_ref[...] = e / jnp.sum(e, axis=1, keepdims=True)


def _tc_head(x, acc0, deg0, acc1, deg1, W_self, W_r0, W_r1, b, W_out, b_out):
    grid = (N // BLK,)
    row = lambda i: (i, 0)
    full = lambda i: (0, 0)
    return pl.pallas_call(
        _tc_body,
        grid=grid,
        in_specs=[
            pl.BlockSpec((BLK, D), row),
            pl.BlockSpec((BLK, D), row),
            pl.BlockSpec((BLK, 1), row),
            pl.BlockSpec((BLK, D), row),
            pl.BlockSpec((BLK, 1), row),
            pl.BlockSpec((D, H), full),
            pl.BlockSpec((D, H), full),
            pl.BlockSpec((D, H), full),
            pl.BlockSpec((1, H), full),
            pl.BlockSpec((H, C), full),
            pl.BlockSpec((1, C), full),
        ],
        out_specs=pl.BlockSpec((BLK, C), row),
        out_shape=jax.ShapeDtypeStruct((N, C), jnp.float32),
    )(x, acc0, deg0, acc1, deg1, W_self, W_r0, W_r1, b, W_out, b_out)


def kernel(x, edge_index_0, edge_index_1, W_self, W_r0, W_r1, b, W_out, b_out):
    acc0, deg0, acc1, deg1 = _sc_aggregate(
        x,
        edge_index_0.reshape(2, NUM_SUBCORES, NBLK, K),
        edge_index_1.reshape(2, NUM_SUBCORES, NBLK, K),
    )
    return _tc_head(
        x, acc0, deg0.reshape(N, 1), acc1, deg1.reshape(N, 1),
        W_self, W_r0, W_r1, b.reshape(1, H), W_out, b_out.reshape(1, C))


# speedup vs baseline: 12.0731x; 1.0253x over previous
import functools

import jax
import jax.numpy as jnp
from jax import lax
from jax.experimental import pallas as pl
from jax.experimental.pallas import tpu as pltpu
from jax.experimental.pallas import tpu_sc as plsc

N = 10000
D = 128
H = 128
C = 64
E = 160000

NUM_CORES = 2
NUM_SUBCORES = 16
EDGES_PER_TILE = E // NUM_SUBCORES
K = 125
NBLK = EDGES_PER_TILE // K
RB = 80
N_RBLK = N // RB
RBLK_PER_TILE = 8


def _sc_aggregate(x, e0, e1):
    mesh = plsc.VectorSubcoreMesh(core_axis_name="c", subcore_axis_name="s")

    @functools.partial(
        pl.kernel,
        out_type=(
            jax.ShapeDtypeStruct((N, D), jnp.float32),
            jax.ShapeDtypeStruct((N,), jnp.float32),
            jax.ShapeDtypeStruct((N, D), jnp.float32),
            jax.ShapeDtypeStruct((N,), jnp.float32),
        ),
        mesh=mesh,
        scratch_types=[
            pltpu.VMEM((4, K), jnp.int32),
            pltpu.VMEM((4, K), jnp.int32),
            pltpu.VMEM((K, D), jnp.float32),
            pltpu.VMEM((K, D), jnp.float32),
            pltpu.VMEM((K,), jnp.float32),
            pltpu.VMEM((RB,), jnp.float32),
            pltpu.VMEM((RB,), jnp.float32),
            pltpu.VMEM_SHARED((N, D), jnp.float32),
            pltpu.VMEM_SHARED((N,), jnp.float32),
            [pltpu.SemaphoreType.DMA] * 4,
            [pltpu.SemaphoreType.DMA] * 2,
            [pltpu.SemaphoreType.DMA] * 2,
            [pltpu.SemaphoreType.DMA] * 2,
        ],
    )
    def sc_kernel(x_hbm, e0_hbm, e1_hbm,
                  acc0_hbm, deg0_hbm, acc1_hbm, deg1_hbm,
                  sidx, didx, rows0, rows1, ones, zdeg, zdeg2,
                  accs, degs, isems, gsems, ssems, dsems):
        cid = lax.axis_index("c")
        sid = lax.axis_index("s")

        zero16 = jnp.zeros((16,), jnp.float32)
        one16 = jnp.ones((16,), jnp.float32)

        def fill_ones(i, _):
            ones[pl.ds(jnp.minimum(i * 16, K - 16), 16)] = one16
            return 0
        lax.fori_loop(0, (K + 15) // 16, fill_ones, 0)

        rowslots = (rows0, rows1)

        def run_relation(e_hbm, acc_hbm, deg_hbm):
            def fill_zdeg(i, _):
                zdeg[pl.ds(i * 16, 16)] = zero16
                return 0
            lax.fori_loop(0, RB // 16, fill_zdeg, 0)

            def fill_zrows(i, _):
                r = i // (D // 16)
                c = (i % (D // 16)) * 16
                rows0[r, pl.ds(c, 16)] = zero16
                return 0
            lax.fori_loop(0, RB * (D // 16), fill_zrows, 0)

            def init_pair(j):
                off = pl.multiple_of((j * NUM_SUBCORES + sid) * RB, RB)
                return (rows0.at[pl.ds(0, RB)], accs.at[pl.ds(off, RB)],
                        zdeg, degs.at[pl.ds(off, RB)])

            for j in range(RBLK_PER_TILE):
                za, da, zd, dd = init_pair(j)
                if j < RBLK_PER_TILE - 1:
                    pltpu.async_copy(za, da, ssems[0])
                    pltpu.async_copy(zd, dd, dsems[0])
                else:
                    @pl.when(j * NUM_SUBCORES + sid < N_RBLK)
                    def _():
                        pltpu.async_copy(za, da, ssems[0])
                        pltpu.async_copy(zd, dd, dsems[0])

            for j in range(RBLK_PER_TILE):
                za, da, zd, dd = init_pair(j)
                if j < RBLK_PER_TILE - 1:
                    pltpu.make_async_copy(za, da, ssems[0]).wait()
                    pltpu.make_async_copy(zd, dd, dsems[0]).wait()
                else:
                    @pl.when(j * NUM_SUBCORES + sid < N_RBLK)
                    def _():
                        pltpu.make_async_copy(za, da, ssems[0]).wait()
                        pltpu.make_async_copy(zd, dd, dsems[0]).wait()

            plsc.subcore_barrier()

            def idx_load(blk, slot):
                pltpu.async_copy(e_hbm.at[0, sid, blk], sidx.at[slot], isems[slot])
                pltpu.async_copy(e_hbm.at[1, sid, blk], didx.at[slot], isems[slot])

            def wait_idx(blk, slot):
                pltpu.make_async_copy(e_hbm.at[0, sid, blk], sidx.at[slot], isems[slot]).wait()
                pltpu.make_async_copy(e_hbm.at[1, sid, blk], didx.at[slot], isems[slot]).wait()

            def gather(rows, slot, g):
                pltpu.async_copy(x_hbm.at[sidx.at[slot]], rows, gsems[g])

            def wait_gather(rows, slot, g):
                pltpu.make_async_copy(x_hbm.at[sidx.at[slot]], rows, gsems[g]).wait()

            for t in range(4):
                idx_load(t, t)
            wait_idx(0, 0)
            gather(rows0, 0, 0)
            wait_idx(1, 1)
            gather(rows1, 1, 1)

            def body(j, _):
                for t in range(4):
                    b = j * 4 + t
                    rows = rowslots[t % 2]
                    g = t % 2
                    wait_gather(rows, t, g)
                    pltpu.async_copy(rows, accs.at[didx.at[t]], ssems[g], add=True)
                    pltpu.async_copy(ones, degs.at[didx.at[t]], dsems[g], add=True)

                    t2 = (t + 2) % 4

                    @pl.when(b + 2 < NBLK)
                    def _():
                        wait_idx(b + 2, t2)

                    pltpu.make_async_copy(rows, accs.at[didx.at[t]], ssems[g]).wait()

                    @pl.when(b + 2 < NBLK)
                    def _():
                        gather(rows, t2, g)

                    pltpu.make_async_copy(ones, degs.at[didx.at[t]], dsems[g]).wait()

                    @pl.when(b + 4 < NBLK)
                    def _():
                        idx_load(b + 4, t)
                return 0
            lax.fori_loop(0, NBLK // 4, body, 0)

            plsc.subcore_barrier()

            wslots = (rows0, rows1)
            dslots = (zdeg, zdeg2)

            def wb_refs(j):
                off = pl.multiple_of((j * NUM_SUBCORES + sid) * RB, RB)
                s = j % 2
                return (accs.at[pl.ds(off, RB)], wslots[s].at[pl.ds(0, RB)],
                        acc_hbm.at[pl.ds(off, RB)],
                        degs.at[pl.ds(off, RB)], dslots[s],
                        deg_hbm.at[pl.ds(off, RB)], s)

            def wb_issue(j):
                asrc, abuf, adst, dsrc, dbuf, ddst, s = wb_refs(j)
                pltpu.sync_copy(asrc, abuf)
                pltpu.async_copy(abuf, adst, gsems[s])
                pltpu.sync_copy(dsrc, dbuf)
                pltpu.async_copy(dbuf, ddst, isems[s])

            def wb_wait(j):
                _, abuf, adst, _, dbuf, ddst, s = wb_refs(j)
                pltpu.make_async_copy(abuf, adst, gsems[s]).wait()
                pltpu.make_async_copy(dbuf, ddst, isems[s]).wait()

            for j in range(RBLK_PER_TILE):
                if j >= 2:
                    wb_wait(j - 2)
                if j < RBLK_PER_TILE - 1:
                    wb_issue(j)
                else:
                    @pl.when(j * NUM_SUBCORES + sid < N_RBLK)
                    def _():
                        wb_issue(j)
            wb_wait(RBLK_PER_TILE - 2)

            @pl.when((RBLK_PER_TILE - 1) * NUM_SUBCORES + sid < N_RBLK)
            def _():
                wb_wait(RBLK_PER_TILE - 1)

        @pl.when(cid == 0)
        def _():
            run_relation(e0_hbm, acc0_hbm, deg0_hbm)

        @pl.when(cid == 1)
        def _():
            run_relation(e1_hbm, acc1_hbm, deg1_hbm)

    return sc_kernel(x, e0, e1)


BLK = 2000


def _tc_body(x_ref, a0_ref, d0_ref, a1_ref, d1_ref,
             ws_ref, w0_ref, w1_ref, b_ref, wo_ref, bo_ref, out_ref):
    xb = x_ref[...]
    r0 = 1.0 / jnp.maximum(d0_ref[...], 1.0)
    r1 = 1.0 / jnp.maximum(d1_ref[...], 1.0)
    h = jnp.dot(xb, ws_ref[...], preferred_element_type=jnp.float32)
    h = h + jnp.dot(a0_ref[...] * r0, w0_ref[...], preferred_element_type=jnp.float32)
    h = h + jnp.dot(a1_ref[...] * r1, w1_ref[...], preferred_element_type=jnp.float32)
    h = jnp.maximum(h + b_ref[...], 0.0)
    logits = jnp.dot(h, wo_ref[...], preferred_element_type=jnp.float32) + bo_ref[...]
    m = jnp.max(logits, axis=1, keepdims=True)
    e = jnp.exp(logits - m)
    out_ref[...] = e / jnp.sum(e, axis=1, keepdims=True)


def _tc_head(x, acc0, deg0, acc1, deg1, W_self, W_r0, W_r1, b, W_out, b_out):
    grid = (N // BLK,)
    row = lambda i: (i, 0)
    full = lambda i: (0, 0)
    return pl.pallas_call(
        _tc_body,
        grid=grid,
        in_specs=[
            pl.BlockSpec((BLK, D), row),
            pl.BlockSpec((BLK, D), row),
            pl.BlockSpec((BLK, 1), row),
            pl.BlockSpec((BLK, D), row),
            pl.BlockSpec((BLK, 1), row),
            pl.BlockSpec((D, H), full),
            pl.BlockSpec((D, H), full),
            pl.BlockSpec((D, H), full),
            pl.BlockSpec((1, H), full),
            pl.BlockSpec((H, C), full),
            pl.BlockSpec((1, C), full),
        ],
        out_specs=pl.BlockSpec((BLK, C), row),
        out_shape=jax.ShapeDtypeStruct((N, C), jnp.float32),
    )(x, acc0, deg0, acc1, deg1, W_self, W_r0, W_r1, b, W_out, b_out)


def kernel(x, edge_index_0, edge_index_1, W_self, W_r0, W_r1, b, W_out, b_out):
    acc0, deg0, acc1, deg1 = _sc_aggregate(
        x,
        edge_index_0.reshape(2, NUM_SUBCORES, NBLK, K),
        edge_index_1.reshape(2, NUM_SUBCORES, NBLK, K),
    )
    return _tc_head(
        x, acc0, deg0.reshape(N, 1), acc1, deg1.reshape(N, 1),
        W_self, W_r0, W_r1, b.reshape(1, H), W_out, b_out.reshape(1, C))


# speedup vs baseline: 12.8597x; 1.0652x over previous
import functools

import jax
import jax.numpy as jnp
from jax import lax
from jax.experimental import pallas as pl
from jax.experimental.pallas import tpu as pltpu
from jax.experimental.pallas import tpu_sc as plsc

N = 10000
D = 128
H = 128
C = 64
E = 160000

NUM_CORES = 2
NUM_SUBCORES = 16
EDGES_PER_TILE = E // NUM_SUBCORES
K = 125
NBLK = EDGES_PER_TILE // K
RB = 80
N_RBLK = N // RB
RBLK_PER_TILE = 8


def _sc_aggregate(x, e0, e1):
    mesh = plsc.VectorSubcoreMesh(core_axis_name="c", subcore_axis_name="s")

    @functools.partial(
        pl.kernel,
        out_type=(
            jax.ShapeDtypeStruct((N, D), jnp.float32),
            jax.ShapeDtypeStruct((N,), jnp.float32),
            jax.ShapeDtypeStruct((N, D), jnp.float32),
            jax.ShapeDtypeStruct((N,), jnp.float32),
        ),
        mesh=mesh,
        scratch_types=[
            pltpu.VMEM((4, K), jnp.int32),
            pltpu.VMEM((4, K), jnp.int32),
            pltpu.VMEM((K, D), jnp.float32),
            pltpu.VMEM((K, D), jnp.float32),
            pltpu.VMEM((K,), jnp.float32),
            pltpu.VMEM((RB,), jnp.float32),
            pltpu.VMEM((RB,), jnp.float32),
            pltpu.VMEM_SHARED((N, D), jnp.float32),
            pltpu.VMEM_SHARED((N,), jnp.float32),
            [pltpu.SemaphoreType.DMA] * 4,
            [pltpu.SemaphoreType.DMA] * 2,
            [pltpu.SemaphoreType.DMA] * 2,
            [pltpu.SemaphoreType.DMA] * 2,
        ],
    )
    def sc_kernel(x_hbm, e0_hbm, e1_hbm,
                  acc0_hbm, deg0_hbm, acc1_hbm, deg1_hbm,
                  sidx, didx, rows0, rows1, ones, zdeg, zdeg2,
                  accs, degs, isems, gsems, ssems, dsems):
        cid = lax.axis_index("c")
        sid = lax.axis_index("s")

        zero16 = jnp.zeros((16,), jnp.float32)
        one16 = jnp.ones((16,), jnp.float32)

        def fill_ones(i, _):
            ones[pl.ds(jnp.minimum(i * 16, K - 16), 16)] = one16
            return 0
        lax.fori_loop(0, (K + 15) // 16, fill_ones, 0)

        rowslots = (rows0, rows1)

        def run_relation(e_hbm, acc_hbm, deg_hbm):
            def fill_zdeg(i, _):
                zdeg[pl.ds(i * 16, 16)] = zero16
                return 0
            lax.fori_loop(0, RB // 16, fill_zdeg, 0)

            def fill_zrows(i, _):
                r = i // (D // 16)
                c = (i % (D // 16)) * 16
                rows0[r, pl.ds(c, 16)] = zero16
                return 0
            lax.fori_loop(0, RB * (D // 16), fill_zrows, 0)

            def init_pair(j):
                off = pl.multiple_of((j * NUM_SUBCORES + sid) * RB, RB)
                return (rows0.at[pl.ds(0, RB)], accs.at[pl.ds(off, RB)],
                        zdeg, degs.at[pl.ds(off, RB)])

            for j in range(RBLK_PER_TILE):
                za, da, zd, dd = init_pair(j)
                if j < RBLK_PER_TILE - 1:
                    pltpu.async_copy(za, da, ssems[0])
                    pltpu.async_copy(zd, dd, dsems[0])
                else:
                    @pl.when(j * NUM_SUBCORES + sid < N_RBLK)
                    def _():
                        pltpu.async_copy(za, da, ssems[0])
                        pltpu.async_copy(zd, dd, dsems[0])

            for j in range(RBLK_PER_TILE):
                za, da, zd, dd = init_pair(j)
                if j < RBLK_PER_TILE - 1:
                    pltpu.make_async_copy(za, da, ssems[0]).wait()
                    pltpu.make_async_copy(zd, dd, dsems[0]).wait()
                else:
                    @pl.when(j * NUM_SUBCORES + sid < N_RBLK)
                    def _():
                        pltpu.make_async_copy(za, da, ssems[0]).wait()
                        pltpu.make_async_copy(zd, dd, dsems[0]).wait()

            plsc.subcore_barrier()

            def idx_load(blk, slot):
                pltpu.async_copy(e_hbm.at[0, sid, blk], sidx.at[slot], isems[slot])
                pltpu.async_copy(e_hbm.at[1, sid, blk], didx.at[slot], isems[slot])

            def wait_idx(blk, slot):
                pltpu.make_async_copy(e_hbm.at[0, sid, blk], sidx.at[slot], isems[slot]).wait()
                pltpu.make_async_copy(e_hbm.at[1, sid, blk], didx.at[slot], isems[slot]).wait()

            def gather(rows, slot, g):
                pltpu.async_copy(x_hbm.at[sidx.at[slot]], rows, gsems[g])

            def wait_gather(rows, slot, g):
                pltpu.make_async_copy(x_hbm.at[sidx.at[slot]], rows, gsems[g]).wait()

            for t in range(4):
                idx_load(t, t)
            wait_idx(0, 0)
            gather(rows0, 0, 0)
            wait_idx(1, 1)
            gather(rows1, 1, 1)

            def body(j, _):
                for t in range(4):
                    b = j * 4 + t
                    rows = rowslots[t % 2]
                    g = t % 2
                    wait_gather(rows, t, g)
                    pltpu.async_copy(rows, accs.at[didx.at[t]], ssems[g], add=True)
                    pltpu.async_copy(ones, degs.at[didx.at[t]], dsems[g], add=True)

                    t2 = (t + 2) % 4

                    @pl.when(b + 2 < NBLK)
                    def _():
                        wait_idx(b + 2, t2)

                    pltpu.make_async_copy(rows, accs.at[didx.at[t]], ssems[g]).wait()

                    @pl.when(b + 2 < NBLK)
                    def _():
                        gather(rows, t2, g)

                    pltpu.make_async_copy(ones, degs.at[didx.at[t]], dsems[g]).wait()

                    @pl.when(b + 4 < NBLK)
                    def _():
                        idx_load(b + 4, t)
                return 0
            lax.fori_loop(0, NBLK // 4, body, 0)

            plsc.subcore_barrier()

            wslots = (rows0, rows1)
            dslots = (zdeg, zdeg2)

            def wb_refs(j):
                off = pl.multiple_of((j * NUM_SUBCORES + sid) * RB, RB)
                s = j % 2
                return (accs.at[pl.ds(off, RB)], wslots[s].at[pl.ds(0, RB)],
                        acc_hbm.at[pl.ds(off, RB)],
                        degs.at[pl.ds(off, RB)], dslots[s],
                        deg_hbm.at[pl.ds(off, RB)], s)

            def wb_issue(j):
                asrc, abuf, adst, dsrc, dbuf, ddst, s = wb_refs(j)
                pltpu.sync_copy(asrc, abuf)
                pltpu.async_copy(abuf, adst, gsems[s])
                pltpu.sync_copy(dsrc, dbuf)
                pltpu.async_copy(dbuf, ddst, isems[s])

            def wb_wait(j):
                _, abuf, adst, _, dbuf, ddst, s = wb_refs(j)
                pltpu.make_async_copy(abuf, adst, gsems[s]).wait()
                pltpu.make_async_copy(dbuf, ddst, isems[s]).wait()

            for j in range(RBLK_PER_TILE):
                if j >= 2:
                    wb_wait(j - 2)
                if j < RBLK_PER_TILE - 1:
                    wb_issue(j)
                else:
                    @pl.when(j * NUM_SUBCORES + sid < N_RBLK)
                    def _():
                        wb_issue(j)
            wb_wait(RBLK_PER_TILE - 2)

            @pl.when((RBLK_PER_TILE - 1) * NUM_SUBCORES + sid < N_RBLK)
            def _():
                wb_wait(RBLK_PER_TILE - 1)

        @pl.when(cid == 0)
        def _():
            run_relation(e0_hbm, acc0_hbm, deg0_hbm)

        @pl.when(cid == 1)
        def _():
            run_relation(e1_hbm, acc1_hbm, deg1_hbm)

    return sc_kernel(x, e0, e1)


BLK = 2048
NPAD = 10240


def _tc_body(x_ref, a0_ref, d0_ref, a1_ref, d1_ref,
             ws_ref, w0_ref, w1_ref, b_ref, wo_ref, bo_ref, out_ref):
    xb = x_ref[...]
    i = pl.program_id(0)
    d0 = d0_ref[:, pl.ds(i * BLK, BLK)]
    d1 = d1_ref[:, pl.ds(i * BLK, BLK)]
    r0 = 1.0 / jnp.maximum(d0.T, 1.0)
    r1 = 1.0 / jnp.maximum(d1.T, 1.0)
    h = jnp.dot(xb, ws_ref[...], preferred_element_type=jnp.float32)
    h = h + jnp.dot(a0_ref[...] * r0, w0_ref[...], preferred_element_type=jnp.float32)
    h = h + jnp.dot(a1_ref[...] * r1, w1_ref[...], preferred_element_type=jnp.float32)
    h = jnp.maximum(h + b_ref[...], 0.0)
    logits = jnp.dot(h, wo_ref[...], preferred_element_type=jnp.float32) + bo_ref[...]
    m = jnp.max(logits, axis=1, keepdims=True)
    e = jnp.exp(logits - m)
    out_ref[...] = e / jnp.sum(e, axis=1, keepdims=True)


def _tc_head(x, acc0, deg0, acc1, deg1, W_self, W_r0, W_r1, b, W_out, b_out):
    grid = (pl.cdiv(N, BLK),)
    row = lambda i: (i, 0)
    full = lambda i: (0, 0)
    return pl.pallas_call(
        _tc_body,
        grid=grid,
        in_specs=[
            pl.BlockSpec((BLK, D), row),
            pl.BlockSpec((BLK, D), row),
            pl.BlockSpec((1, NPAD), full),
            pl.BlockSpec((BLK, D), row),
            pl.BlockSpec((1, NPAD), full),
            pl.BlockSpec((D, H), full),
            pl.BlockSpec((D, H), full),
            pl.BlockSpec((D, H), full),
            pl.BlockSpec((1, H), full),
            pl.BlockSpec((H, C), full),
            pl.BlockSpec((1, C), full),
        ],
        out_specs=pl.BlockSpec((BLK, C), row),
        out_shape=jax.ShapeDtypeStruct((N, C), jnp.float32),
    )(x, acc0, deg0, acc1, deg1, W_self, W_r0, W_r1, b, W_out, b_out)


def kernel(x, edge_index_0, edge_index_1, W_self, W_r0, W_r1, b, W_out, b_out):
    acc0, deg0, acc1, deg1 = _sc_aggregate(
        x,
        edge_index_0.reshape(2, NUM_SUBCORES, NBLK, K),
        edge_index_1.reshape(2, NUM_SUBCORES, NBLK, K),
    )
    return _tc_head(
        x,
        acc0, jnp.pad(deg0, (0, NPAD - N)).reshape(1, NPAD),
        acc1, jnp.pad(deg1, (0, NPAD - N)).reshape(1, NPAD),
        W_self, W_r0, W_r1, b.reshape(1, H), W_out, b_out.reshape(1, C))
